# baseline (speedup 1.0000x reference)
import functools

import jax
import jax.numpy as jnp
from jax import lax
from jax.experimental import pallas as pl
from jax.experimental.pallas import tpu as pltpu
from jax.experimental.pallas import tpu_sc as plsc

N = 10000
E = 320000
D = 128

NC = 2
NS = 16
EDGES_PER_TILE = E // (NC * NS)

_mesh = plsc.VectorSubcoreMesh(core_axis_name="c", subcore_axis_name="s")


def _pipelined_edge_loop(nch, load_idx, start_gather, wait_gather,
                         start_scatter, wait_scatter):
    load_idx(0, 0)
    start_gather(0)

    def body(j, carry):
        wait_gather(0)
        start_scatter(0)

        @pl.when(j > 0)
        def _():
            wait_scatter(1)

        load_idx(2 * j + 1, 1)
        start_gather(1)
        wait_gather(1)
        start_scatter(1)
        wait_scatter(0)

        @pl.when(2 * j + 2 < nch)
        def _():
            load_idx(2 * j + 2, 0)
            start_gather(0)

        return carry

    lax.fori_loop(0, nch // 2, body, 0)
    if nch % 2:
        wait_gather(0)
        start_scatter(0)
        wait_scatter(0)
    wait_scatter(1)


DEG_CHUNK = 400
DEG_NCHUNKS = EDGES_PER_TILE // DEG_CHUNK


@functools.partial(
    pl.kernel,
    mesh=_mesh,
    out_type=jax.ShapeDtypeStruct((NC, N), jnp.float32),
    scratch_types=[
        pltpu.VMEM((DEG_CHUNK,), jnp.int32),
        pltpu.VMEM((DEG_CHUNK,), jnp.float32),
        pltpu.VMEM_SHARED((N,), jnp.float32),
    ],
)
def _deg_kernel(dst_hbm, z_hbm, out_hbm, dst_v, ones_v, acc_sh):
    c = lax.axis_index("c")
    s = lax.axis_index("s")

    @pl.when(s == 0)
    def _():
        pltpu.sync_copy(z_hbm, acc_sh)

    def fill_ones(j, carry):
        ones_v[pl.ds(j * 16, 16)] = jnp.ones((16,), jnp.float32)
        return carry

    lax.fori_loop(0, DEG_CHUNK // 16, fill_ones, 0)
    plsc.subcore_barrier()

    base0 = (c * NS + s) * EDGES_PER_TILE

    def chunk(k, carry):
        base = base0 + k * DEG_CHUNK
        pltpu.sync_copy(dst_hbm.at[pl.ds(base, DEG_CHUNK)], dst_v)
        pltpu.sync_copy(ones_v, acc_sh.at[dst_v], add=True)
        return carry

    lax.fori_loop(0, DEG_NCHUNKS, chunk, 0)
    plsc.subcore_barrier()

    @pl.when(s == 0)
    def _():
        pltpu.sync_copy(acc_sh, out_hbm.at[c])


PC = 80
PNCH = EDGES_PER_TILE // PC


@functools.partial(
    pl.kernel,
    mesh=_mesh,
    out_type=jax.ShapeDtypeStruct((NC, N, D), jnp.float32),
    scratch_types=[
        pltpu.VMEM((PC,), jnp.int32),
        pltpu.VMEM((PC,), jnp.int32),
        pltpu.VMEM((PC,), jnp.int32),
        pltpu.VMEM((PC,), jnp.int32),
        pltpu.VMEM((PC, D), jnp.float32),
        pltpu.VMEM((PC, D), jnp.float32),
        pltpu.VMEM_SHARED((N, D), jnp.float32),
        pltpu.SemaphoreType.DMA,
        pltpu.SemaphoreType.DMA,
        pltpu.SemaphoreType.DMA,
        pltpu.SemaphoreType.DMA,
    ],
)
def _prop_kernel(g_hbm, src_hbm, dst_hbm, z_hbm, out_hbm,
                 src_v0, src_v1, dst_v0, dst_v1, rows_v0, rows_v1, acc_sh,
                 sg0, sg1, ss0, ss1):
    c = lax.axis_index("c")
    s = lax.axis_index("s")

    @pl.when(s == 0)
    def _():
        pltpu.sync_copy(z_hbm, acc_sh)

    plsc.subcore_barrier()

    base0 = (c * NS + s) * EDGES_PER_TILE
    src_v = [src_v0, src_v1]
    dst_v = [dst_v0, dst_v1]
    rows_v = [rows_v0, rows_v1]
    sg = [sg0, sg1]
    ss = [ss0, ss1]

    def load_idx(k, b):
        base = base0 + k * PC
        pltpu.sync_copy(src_hbm.at[pl.ds(base, PC)], src_v[b])
        pltpu.sync_copy(dst_hbm.at[pl.ds(base, PC)], dst_v[b])

    def start_gather(b):
        pltpu.async_copy(g_hbm.at[src_v[b]], rows_v[b], sg[b])

    def wait_gather(b):
        pltpu.make_async_copy(g_hbm.at[src_v[b]], rows_v[b], sg[b]).wait()

    def start_scatter(b):
        pltpu.async_copy(rows_v[b], acc_sh.at[dst_v[b]], ss[b], add=True)

    def wait_scatter(b):
        pltpu.make_async_copy(rows_v[b], acc_sh.at[dst_v[b]], ss[b]).wait()

    _pipelined_edge_loop(PNCH, load_idx, start_gather, wait_gather,
                         start_scatter, wait_scatter)

    plsc.subcore_barrier()

    @pl.when(s == 0)
    def _():
        pltpu.sync_copy(acc_sh, out_hbm.at[c])


P1C = 400
P1NCH = EDGES_PER_TILE // P1C


@functools.partial(
    pl.kernel,
    mesh=_mesh,
    out_type=jax.ShapeDtypeStruct((NC, N), jnp.float32),
    scratch_types=[
        pltpu.VMEM((P1C,), jnp.int32),
        pltpu.VMEM((P1C,), jnp.int32),
        pltpu.VMEM((P1C,), jnp.int32),
        pltpu.VMEM((P1C,), jnp.int32),
        pltpu.VMEM((P1C,), jnp.float32),
        pltpu.VMEM((P1C,), jnp.float32),
        pltpu.VMEM_SHARED((N,), jnp.float32),
        pltpu.SemaphoreType.DMA,
        pltpu.SemaphoreType.DMA,
        pltpu.SemaphoreType.DMA,
        pltpu.SemaphoreType.DMA,
    ],
)
def _prop1_kernel(g_hbm, src_hbm, dst_hbm, z_hbm, out_hbm,
                  src_v0, src_v1, dst_v0, dst_v1, vals_v0, vals_v1, acc_sh,
                  sg0, sg1, ss0, ss1):
    c = lax.axis_index("c")
    s = lax.axis_index("s")

    @pl.when(s == 0)
    def _():
        pltpu.sync_copy(z_hbm, acc_sh)

    plsc.subcore_barrier()

    base0 = (c * NS + s) * EDGES_PER_TILE
    src_v = [src_v0, src_v1]
    dst_v = [dst_v0, dst_v1]
    vals_v = [vals_v0, vals_v1]
    sg = [sg0, sg1]
    ss = [ss0, ss1]

    def load_idx(k, b):
        base = base0 + k * P1C
        pltpu.sync_copy(src_hbm.at[pl.ds(base, P1C)], src_v[b])
        pltpu.sync_copy(dst_hbm.at[pl.ds(base, P1C)], dst_v[b])

    def start_gather(b):
        pltpu.async_copy(g_hbm.at[src_v[b]], vals_v[b], sg[b])

    def wait_gather(b):
        pltpu.make_async_copy(g_hbm.at[src_v[b]], vals_v[b], sg[b]).wait()

    def start_scatter(b):
        pltpu.async_copy(vals_v[b], acc_sh.at[dst_v[b]], ss[b], add=True)

    def wait_scatter(b):
        pltpu.make_async_copy(vals_v[b], acc_sh.at[dst_v[b]], ss[b]).wait()

    _pipelined_edge_loop(P1NCH, load_idx, start_gather, wait_gather,
                         start_scatter, wait_scatter)

    plsc.subcore_barrier()

    @pl.when(s == 0)
    def _():
        pltpu.sync_copy(acc_sh, out_hbm.at[c])


ROWS = 1000


def _tc1_body(degp_ref, x_ref, w_ref, g_ref, dinv_ref):
    d = degp_ref[0] + degp_ref[1] + 1.0
    dinv = lax.rsqrt(d)
    dinv_ref[...] = dinv
    h = jnp.dot(x_ref[...], w_ref[...], preferred_element_type=jnp.float32)
    g_ref[...] = h * dinv


def _tc_layer_body(p_ref, g_ref, dinv_ref, b_ref, w_ref, out_ref):
    s = p_ref[0] + p_ref[1] + g_ref[...]
    a = jnp.maximum(s * dinv_ref[...] + b_ref[...], 0.0)
    h = jnp.dot(a, w_ref[...], preferred_element_type=jnp.float32)
    out_ref[...] = h * dinv_ref[...]


def _tc_out_body(p_ref, g_ref, dinv_ref, b_ref, out_ref):
    out_ref[...] = (p_ref[0] + p_ref[1] + g_ref[...]) * dinv_ref[...] + b_ref[...]


def _tc1(degp, x, w0):
    return pl.pallas_call(
        _tc1_body,
        grid=(N // ROWS,),
        in_specs=[
            pl.BlockSpec((NC, ROWS, 1), lambda i: (0, i, 0)),
            pl.BlockSpec((ROWS, D), lambda i: (i, 0)),
            pl.BlockSpec((D, D), lambda i: (0, 0)),
        ],
        out_specs=[
            pl.BlockSpec((ROWS, D), lambda i: (i, 0)),
            pl.BlockSpec((ROWS, 1), lambda i: (i, 0)),
        ],
        out_shape=[
            jax.ShapeDtypeStruct((N, D), jnp.float32),
            jax.ShapeDtypeStruct((N, 1), jnp.float32),
        ],
    )(degp, x, w0)


def _tc_layer(p, g, dinv, b, w):
    dout = w.shape[1]
    return pl.pallas_call(
        _tc_layer_body,
        grid=(N // ROWS,),
        in_specs=[
            pl.BlockSpec((NC, ROWS, D), lambda i: (0, i, 0)),
            pl.BlockSpec((ROWS, D), lambda i: (i, 0)),
            pl.BlockSpec((ROWS, 1), lambda i: (i, 0)),
            pl.BlockSpec((1, D), lambda i: (0, 0)),
            pl.BlockSpec((D, dout), lambda i: (0, 0)),
        ],
        out_specs=pl.BlockSpec((ROWS, dout), lambda i: (i, 0)),
        out_shape=jax.ShapeDtypeStruct((N, dout), jnp.float32),
    )(p, g, dinv, b, w)


def _tc_out(p, g2, dinv, b2):
    return pl.pallas_call(
        _tc_out_body,
        grid=(N // ROWS,),
        in_specs=[
            pl.BlockSpec((NC, ROWS, 1), lambda i: (0, i, 0)),
            pl.BlockSpec((ROWS, 1), lambda i: (i, 0)),
            pl.BlockSpec((ROWS, 1), lambda i: (i, 0)),
            pl.BlockSpec((1, 1), lambda i: (0, 0)),
        ],
        out_specs=pl.BlockSpec((ROWS, 1), lambda i: (i, 0)),
        out_shape=jax.ShapeDtypeStruct((N, 1), jnp.float32),
    )(p, g2, dinv, b2)


def kernel(x, edge_index, W0, b0, W1, b1, W2, b2):
    src = edge_index[0].astype(jnp.int32)
    dst = edge_index[1].astype(jnp.int32)
    zN = jnp.zeros((N,), jnp.float32)
    zND = jnp.zeros((N, D), jnp.float32)

    degp = _deg_kernel(dst, zN)
    g0, dinv = _tc1(degp.reshape(NC, N, 1), x, W0)
    p0 = _prop_kernel(g0, src, dst, zND)
    g1 = _tc_layer(p0, g0, dinv, b0.reshape(1, D), W1)
    p1 = _prop_kernel(g1, src, dst, zND)
    g2 = _tc_layer(p1, g1, dinv, b1.reshape(1, D), W2)
    p2 = _prop1_kernel(g2.reshape(N), src, dst, zN)
    out = _tc_out(p2.reshape(NC, N, 1), g2, dinv, b2.reshape(1, 1))
    return out


# speedup vs baseline: 1.8616x; 1.8616x over previous
import functools

import jax
import jax.numpy as jnp
from jax import lax
from jax.experimental import pallas as pl
from jax.experimental.pallas import tpu as pltpu
from jax.experimental.pallas import tpu_sc as plsc

N = 10000
E = 320000
D = 128

NC = 2
NS = 16
NP = N + 16
EPT = 10240
EP = EPT * NC * NS

_mesh = plsc.VectorSubcoreMesh(core_axis_name="c", subcore_axis_name="s")


def _ring3_edge_loop(nch, src_load, src_wait, dst_load, dst_wait,
                     g_start, g_wait, s_start, s_wait):
    def make_iter(k, b, static):
        def when(cond, fn):
            if static:
                if cond:
                    fn()
            else:
                pl.when(cond)(fn)

        when(k >= 1 if static else k >= 1, lambda: s_wait((b + 2) % 3))
        when(k + 2 < nch, lambda: dst_load(k + 2, (b + 2) % 3))
        when(k + 2 < nch, lambda: src_wait((b + 2) % 3))
        when(k + 2 < nch, lambda: g_start(k + 2, (b + 2) % 3))
        g_wait(b)
        when(k + 3 < nch, lambda: src_load(k + 3, b))
        dst_wait(b)
        s_start(k, b)

    src_load(0, 0)
    dst_load(0, 0)
    src_load(1, 1)
    dst_load(1, 1)
    src_wait(0)
    src_wait(1)
    g_start(0, 0)
    g_start(1, 1)
    if nch > 2:
        src_load(2, 2)

    ngroups = nch // 3

    def body(j, carry):
        for t in range(3):
            make_iter(3 * j + t, t, static=False)
        return carry

    lax.fori_loop(0, ngroups, body, 0)
    for k in range(3 * ngroups, nch):
        make_iter(k, k % 3, static=True)
    s_wait((nch - 1) % 3)


def _edge_scatter_kernel(gv, src_hbm, dst_hbm, z_hbm, out_hbm,
                         src_v, dst_v, rows_v, sr, sd, sg, ss,
                         acc_sh, pc, nch, c, s):
    @pl.when(s == 0)
    def _():
        pltpu.sync_copy(z_hbm, acc_sh)

    plsc.subcore_barrier()

    base0 = (c * NS + s) * EPT

    def src_load(k, b):
        pltpu.async_copy(src_hbm.at[pl.ds(base0 + k * pc, pc)], src_v[b], sr[b])

    def src_wait(b):
        pltpu.make_async_copy(src_hbm.at[pl.ds(base0, pc)], src_v[b], sr[b]).wait()

    def dst_load(k, b):
        pltpu.async_copy(dst_hbm.at[pl.ds(base0 + k * pc, pc)], dst_v[b], sd[b])

    def dst_wait(b):
        pltpu.make_async_copy(dst_hbm.at[pl.ds(base0, pc)], dst_v[b], sd[b]).wait()

    def g_start(k, b):
        pltpu.async_copy(gv.at[src_v[b]], rows_v[b], sg[b])

    def g_wait(b):
        pltpu.make_async_copy(gv.at[src_v[b]], rows_v[b], sg[b]).wait()

    def s_start(k, b):
        pltpu.async_copy(rows_v[b], acc_sh.at[dst_v[b]], ss[b], add=True)

    def s_wait(b):
        pltpu.make_async_copy(rows_v[b], acc_sh.at[dst_v[b]], ss[b]).wait()

    _ring3_edge_loop(nch, src_load, src_wait, dst_load, dst_wait,
                     g_start, g_wait, s_start, s_wait)

    plsc.subcore_barrier()

    @pl.when(s == 0)
    def _():
        pltpu.sync_copy(acc_sh, out_hbm.at[c])


DEG_CHUNK = 512
DEG_NCHUNKS = EPT // DEG_CHUNK


@functools.partial(
    pl.kernel,
    mesh=_mesh,
    out_type=jax.ShapeDtypeStruct((NC, NP), jnp.float32),
    scratch_types=[
        pltpu.VMEM((DEG_CHUNK,), jnp.int32),
        pltpu.VMEM((DEG_CHUNK,), jnp.float32),
        pltpu.VMEM_SHARED((NP,), jnp.float32),
    ],
)
def _deg_kernel(dst_hbm, z_hbm, out_hbm, dst_v, ones_v, acc_sh):
    c = lax.axis_index("c")
    s = lax.axis_index("s")

    @pl.when(s == 0)
    def _():
        pltpu.sync_copy(z_hbm, acc_sh)

    def fill_ones(j, carry):
        ones_v[pl.ds(j * 16, 16)] = jnp.ones((16,), jnp.float32)
        return carry

    lax.fori_loop(0, DEG_CHUNK // 16, fill_ones, 0)
    plsc.subcore_barrier()

    base0 = (c * NS + s) * EPT

    def chunk(k, carry):
        base = base0 + k * DEG_CHUNK
        pltpu.sync_copy(dst_hbm.at[pl.ds(base, DEG_CHUNK)], dst_v)
        pltpu.sync_copy(ones_v, acc_sh.at[dst_v], add=True)
        return carry

    lax.fori_loop(0, DEG_NCHUNKS, chunk, 0)
    plsc.subcore_barrier()

    @pl.when(s == 0)
    def _():
        pltpu.sync_copy(acc_sh, out_hbm.at[c])


PC = 128
PNCH = EPT // PC


@functools.partial(
    pl.kernel,
    mesh=_mesh,
    out_type=jax.ShapeDtypeStruct((NC, NP, D), jnp.float32),
    scratch_types=(
        [pltpu.VMEM((PC,), jnp.int32)] * 3
        + [pltpu.VMEM((PC,), jnp.int32)] * 3
        + [pltpu.VMEM((PC, D), jnp.float32)] * 3
        + [pltpu.VMEM_SHARED((NP, D), jnp.float32)]
        + [pltpu.SemaphoreType.DMA] * 12
    ),
)
def _prop_kernel(g_hbm, src_hbm, dst_hbm, z_hbm, out_hbm,
                 s0, s1, s2, d0, d1, d2, r0, r1, r2, acc_sh, *sems):
    c = lax.axis_index("c")
    s = lax.axis_index("s")
    _edge_scatter_kernel(g_hbm, src_hbm, dst_hbm, z_hbm, out_hbm,
                         [s0, s1, s2], [d0, d1, d2], [r0, r1, r2],
                         list(sems[0:3]), list(sems[3:6]), list(sems[6:9]),
                         list(sems[9:12]), acc_sh, PC, PNCH, c, s)


P1C = 512
P1NCH = EPT // P1C


@functools.partial(
    pl.kernel,
    mesh=_mesh,
    out_type=jax.ShapeDtypeStruct((NC, NP), jnp.float32),
    scratch_types=(
        [pltpu.VMEM((P1C,), jnp.int32)] * 3
        + [pltpu.VMEM((P1C,), jnp.int32)] * 3
        + [pltpu.VMEM((P1C,), jnp.float32)] * 3
        + [pltpu.VMEM_SHARED((NP,), jnp.float32)]
        + [pltpu.SemaphoreType.DMA] * 12
    ),
)
def _prop1_kernel(g_hbm, src_hbm, dst_hbm, z_hbm, out_hbm,
                  s0, s1, s2, d0, d1, d2, r0, r1, r2, acc_sh, *sems):
    c = lax.axis_index("c")
    s = lax.axis_index("s")
    _edge_scatter_kernel(g_hbm, src_hbm, dst_hbm, z_hbm, out_hbm,
                         [s0, s1, s2], [d0, d1, d2], [r0, r1, r2],
                         list(sems[0:3]), list(sems[3:6]), list(sems[6:9]),
                         list(sems[9:12]), acc_sh, P1C, P1NCH, c, s)


ROWS = 1000


def _tc1_body(degp_ref, x_ref, w_ref, g_ref, dinv_ref):
    d = degp_ref[0] + degp_ref[1] + 1.0
    dinv = lax.rsqrt(d)
    dinv_ref[...] = dinv
    h = jnp.dot(x_ref[...], w_ref[...], preferred_element_type=jnp.float32)
    g_ref[...] = h * dinv


def _tc_layer_body(p_ref, g_ref, dinv_ref, b_ref, w_ref, out_ref):
    s = p_ref[0] + p_ref[1] + g_ref[...]
    a = jnp.maximum(s * dinv_ref[...] + b_ref[...], 0.0)
    h = jnp.dot(a, w_ref[...], preferred_element_type=jnp.float32)
    out_ref[...] = h * dinv_ref[...]


def _tc_out_body(p_ref, g_ref, dinv_ref, b_ref, out_ref):
    out_ref[...] = (p_ref[0] + p_ref[1] + g_ref[...]) * dinv_ref[...] + b_ref[...]


def _tc1(degp, x, w0):
    return pl.pallas_call(
        _tc1_body,
        grid=(N // ROWS,),
        in_specs=[
            pl.BlockSpec((NC, ROWS, 1), lambda i: (0, i, 0)),
            pl.BlockSpec((ROWS, D), lambda i: (i, 0)),
            pl.BlockSpec((D, D), lambda i: (0, 0)),
        ],
        out_specs=[
            pl.BlockSpec((ROWS, D), lambda i: (i, 0)),
            pl.BlockSpec((ROWS, 1), lambda i: (i, 0)),
        ],
        out_shape=[
            jax.ShapeDtypeStruct((NP, D), jnp.float32),
            jax.ShapeDtypeStruct((N, 1), jnp.float32),
        ],
    )(degp, x, w0)


def _tc_layer(p, g, dinv, b, w):
    dout = w.shape[1]
    out_rows = NP if dout == D else N
    return pl.pallas_call(
        _tc_layer_body,
        grid=(N // ROWS,),
        in_specs=[
            pl.BlockSpec((NC, ROWS, D), lambda i: (0, i, 0)),
            pl.BlockSpec((ROWS, D), lambda i: (i, 0)),
            pl.BlockSpec((ROWS, 1), lambda i: (i, 0)),
            pl.BlockSpec((1, D), lambda i: (0, 0)),
            pl.BlockSpec((D, dout), lambda i: (0, 0)),
        ],
        out_specs=pl.BlockSpec((ROWS, dout), lambda i: (i, 0)),
        out_shape=jax.ShapeDtypeStruct((out_rows, dout), jnp.float32),
    )(p, g, dinv, b, w)


def _tc_out(p, g2, dinv, b2):
    return pl.pallas_call(
        _tc_out_body,
        grid=(N // ROWS,),
        in_specs=[
            pl.BlockSpec((NC, ROWS, 1), lambda i: (0, i, 0)),
            pl.BlockSpec((ROWS, 1), lambda i: (i, 0)),
            pl.BlockSpec((ROWS, 1), lambda i: (i, 0)),
            pl.BlockSpec((1, 1), lambda i: (0, 0)),
        ],
        out_specs=pl.BlockSpec((ROWS, 1), lambda i: (i, 0)),
        out_shape=jax.ShapeDtypeStruct((N, 1), jnp.float32),
    )(p, g2, dinv, b2)


def kernel(x, edge_index, W0, b0, W1, b1, W2, b2):
    src = edge_index[0].astype(jnp.int32)
    dst = edge_index[1].astype(jnp.int32)
    pad = (jnp.arange(EP - E, dtype=jnp.int32) % 16) + N
    srcp = jnp.concatenate([src, pad])
    dstp = jnp.concatenate([dst, pad])
    zNP = jnp.zeros((NP,), jnp.float32)
    zNPD = jnp.zeros((NP, D), jnp.float32)

    degp = _deg_kernel(dstp, zNP)
    g0, dinv = _tc1(degp.reshape(NC, NP, 1), x, W0)
    p0 = _prop_kernel(g0, srcp, dstp, zNPD)
    g1 = _tc_layer(p0, g0, dinv, b0.reshape(1, D), W1)
    p1 = _prop_kernel(g1, srcp, dstp, zNPD)
    g2 = _tc_layer(p1, g1, dinv, b1.reshape(1, D), W2)
    g2p = jnp.concatenate([g2.reshape(N), jnp.zeros((16,), jnp.float32)])
    p2 = _prop1_kernel(g2p, srcp, dstp, zNP)
    out = _tc_out(p2.reshape(NC, NP, 1), g2, dinv, b2.reshape(1, 1))
    return out


# speedup vs baseline: 1.8704x; 1.0047x over previous
import functools

import jax
import jax.numpy as jnp
from jax import lax
from jax.experimental import pallas as pl
from jax.experimental.pallas import tpu as pltpu
from jax.experimental.pallas import tpu_sc as plsc

N = 10000
E = 320000
D = 128

NC = 2
NS = 16
NP = N + 112
RSPL = NP // NS
EPT = 10240
EP = EPT * NC * NS

_mesh = plsc.VectorSubcoreMesh(core_axis_name="c", subcore_axis_name="s")


def _ring3_edge_loop(nch, src_load, src_wait, dst_load, dst_wait,
                     g_start, g_wait, s_start, s_wait):
    def make_iter(k, b, static):
        def when(cond, fn):
            if static:
                if cond:
                    fn()
            else:
                pl.when(cond)(fn)

        when(k >= 1 if static else k >= 1, lambda: s_wait((b + 2) % 3))
        when(k + 2 < nch, lambda: dst_load(k + 2, (b + 2) % 3))
        when(k + 2 < nch, lambda: src_wait((b + 2) % 3))
        when(k + 2 < nch, lambda: g_start(k + 2, (b + 2) % 3))
        g_wait(b)
        when(k + 3 < nch, lambda: src_load(k + 3, b))
        dst_wait(b)
        s_start(k, b)

    src_load(0, 0)
    dst_load(0, 0)
    src_load(1, 1)
    dst_load(1, 1)
    src_wait(0)
    src_wait(1)
    g_start(0, 0)
    g_start(1, 1)
    if nch > 2:
        src_load(2, 2)

    ngroups = nch // 3

    def body(j, carry):
        for t in range(3):
            make_iter(3 * j + t, t, static=False)
        return carry

    lax.fori_loop(0, ngroups, body, 0)
    for k in range(3 * ngroups, nch):
        make_iter(k, k % 3, static=True)
    s_wait((nch - 1) % 3)


def _edge_scatter_kernel(gv, src_hbm, dst_hbm, z_hbm, out_hbm,
                         src_v, dst_v, rows_v, sr, sd, sg, ss,
                         acc_sh, pc, nch, c, s, split_io):
    if split_io:
        pltpu.sync_copy(z_hbm.at[pl.ds(s * RSPL, RSPL)],
                        acc_sh.at[pl.ds(s * RSPL, RSPL)])
    else:
        @pl.when(s == 0)
        def _():
            pltpu.sync_copy(z_hbm, acc_sh)
    plsc.subcore_barrier()

    base0 = (c * NS + s) * EPT

    def src_load(k, b):
        pltpu.async_copy(src_hbm.at[pl.ds(base0 + k * pc, pc)], src_v[b], sr[b])

    def src_wait(b):
        pltpu.make_async_copy(src_hbm.at[pl.ds(base0, pc)], src_v[b], sr[b]).wait()

    def dst_load(k, b):
        pltpu.async_copy(dst_hbm.at[pl.ds(base0 + k * pc, pc)], dst_v[b], sd[b])

    def dst_wait(b):
        pltpu.make_async_copy(dst_hbm.at[pl.ds(base0, pc)], dst_v[b], sd[b]).wait()

    def g_start(k, b):
        pltpu.async_copy(gv.at[src_v[b]], rows_v[b], sg[b])

    def g_wait(b):
        pltpu.make_async_copy(gv.at[src_v[b]], rows_v[b], sg[b]).wait()

    def s_start(k, b):
        pltpu.async_copy(rows_v[b], acc_sh.at[dst_v[b]], ss[b], add=True)

    def s_wait(b):
        pltpu.make_async_copy(rows_v[b], acc_sh.at[dst_v[b]], ss[b]).wait()

    _ring3_edge_loop(nch, src_load, src_wait, dst_load, dst_wait,
                     g_start, g_wait, s_start, s_wait)

    plsc.subcore_barrier()
    if split_io:
        pltpu.sync_copy(acc_sh.at[pl.ds(s * RSPL, RSPL)],
                        out_hbm.at[c].at[pl.ds(s * RSPL, RSPL)])
    else:
        @pl.when(s == 0)
        def _():
            pltpu.sync_copy(acc_sh, out_hbm.at[c])


DEG_CHUNK = 512
DEG_NCHUNKS = EPT // DEG_CHUNK


@functools.partial(
    pl.kernel,
    mesh=_mesh,
    out_type=jax.ShapeDtypeStruct((NC, NP), jnp.float32),
    scratch_types=[
        pltpu.VMEM((DEG_CHUNK,), jnp.int32),
        pltpu.VMEM((DEG_CHUNK,), jnp.float32),
        pltpu.VMEM_SHARED((NP,), jnp.float32),
    ],
)
def _deg_kernel(dst_hbm, z_hbm, out_hbm, dst_v, ones_v, acc_sh):
    c = lax.axis_index("c")
    s = lax.axis_index("s")

    @pl.when(s == 0)
    def _():
        pltpu.sync_copy(z_hbm, acc_sh)

    def fill_ones(j, carry):
        ones_v[pl.ds(j * 16, 16)] = jnp.ones((16,), jnp.float32)
        return carry

    lax.fori_loop(0, DEG_CHUNK // 16, fill_ones, 0)
    plsc.subcore_barrier()

    base0 = (c * NS + s) * EPT

    def chunk(k, carry):
        base = base0 + k * DEG_CHUNK
        pltpu.sync_copy(dst_hbm.at[pl.ds(base, DEG_CHUNK)], dst_v)
        pltpu.sync_copy(ones_v, acc_sh.at[dst_v], add=True)
        return carry

    lax.fori_loop(0, DEG_NCHUNKS, chunk, 0)
    plsc.subcore_barrier()

    @pl.when(s == 0)
    def _():
        pltpu.sync_copy(acc_sh, out_hbm.at[c])


PC = 128
PNCH = EPT // PC


@functools.partial(
    pl.kernel,
    mesh=_mesh,
    out_type=jax.ShapeDtypeStruct((NC, NP, D), jnp.float32),
    scratch_types=(
        [pltpu.VMEM((PC,), jnp.int32)] * 3
        + [pltpu.VMEM((PC,), jnp.int32)] * 3
        + [pltpu.VMEM((PC, D), jnp.float32)] * 3
        + [pltpu.VMEM_SHARED((NP, D), jnp.float32)]
        + [pltpu.SemaphoreType.DMA] * 12
    ),
)
def _prop_kernel(g_hbm, src_hbm, dst_hbm, z_hbm, out_hbm,
                 s0, s1, s2, d0, d1, d2, r0, r1, r2, acc_sh, *sems):
    c = lax.axis_index("c")
    s = lax.axis_index("s")
    _edge_scatter_kernel(g_hbm, src_hbm, dst_hbm, z_hbm, out_hbm,
                         [s0, s1, s2], [d0, d1, d2], [r0, r1, r2],
                         list(sems[0:3]), list(sems[3:6]), list(sems[6:9]),
                         list(sems[9:12]), acc_sh, PC, PNCH, c, s, split_io=True)


P1C = 1024
P1NCH = EPT // P1C


@functools.partial(
    pl.kernel,
    mesh=_mesh,
    out_type=jax.ShapeDtypeStruct((NC, NP), jnp.float32),
    scratch_types=(
        [pltpu.VMEM((P1C,), jnp.int32)] * 3
        + [pltpu.VMEM((P1C,), jnp.int32)] * 3
        + [pltpu.VMEM((P1C,), jnp.float32)] * 3
        + [pltpu.VMEM_SHARED((NP,), jnp.float32)]
        + [pltpu.SemaphoreType.DMA] * 12
    ),
)
def _prop1_kernel(g_hbm, src_hbm, dst_hbm, z_hbm, out_hbm,
                  s0, s1, s2, d0, d1, d2, r0, r1, r2, acc_sh, *sems):
    c = lax.axis_index("c")
    s = lax.axis_index("s")
    _edge_scatter_kernel(g_hbm, src_hbm, dst_hbm, z_hbm, out_hbm,
                         [s0, s1, s2], [d0, d1, d2], [r0, r1, r2],
                         list(sems[0:3]), list(sems[3:6]), list(sems[6:9]),
                         list(sems[9:12]), acc_sh, P1C, P1NCH, c, s, split_io=False)


ROWS = 1000


def _tc1_body(degp_ref, x_ref, w_ref, g_ref, dinv_ref):
    d = degp_ref[0] + degp_ref[1] + 1.0
    dinv = lax.rsqrt(d)
    dinv_ref[...] = dinv
    h = jnp.dot(x_ref[...], w_ref[...], preferred_element_type=jnp.float32)
    g_ref[...] = h * dinv


def _tc_layer_body(p_ref, g_ref, dinv_ref, b_ref, w_ref, out_ref):
    s = p_ref[0] + p_ref[1] + g_ref[...]
    a = jnp.maximum(s * dinv_ref[...] + b_ref[...], 0.0)
    h = jnp.dot(a, w_ref[...], preferred_element_type=jnp.float32)
    out_ref[...] = h * dinv_ref[...]


def _tc_out_body(p_ref, g_ref, dinv_ref, b_ref, out_ref):
    out_ref[...] = (p_ref[0] + p_ref[1] + g_ref[...]) * dinv_ref[...] + b_ref[...]


def _tc1(degp, x, w0):
    return pl.pallas_call(
        _tc1_body,
        grid=(N // ROWS,),
        in_specs=[
            pl.BlockSpec((NC, ROWS, 1), lambda i: (0, i, 0)),
            pl.BlockSpec((ROWS, D), lambda i: (i, 0)),
            pl.BlockSpec((D, D), lambda i: (0, 0)),
        ],
        out_specs=[
            pl.BlockSpec((ROWS, D), lambda i: (i, 0)),
            pl.BlockSpec((ROWS, 1), lambda i: (i, 0)),
        ],
        out_shape=[
            jax.ShapeDtypeStruct((NP, D), jnp.float32),
            jax.ShapeDtypeStruct((N, 1), jnp.float32),
        ],
    )(degp, x, w0)


def _tc_layer(p, g, dinv, b, w):
    dout = w.shape[1]
    out_rows = NP if dout == D else N
    return pl.pallas_call(
        _tc_layer_body,
        grid=(N // ROWS,),
        in_specs=[
            pl.BlockSpec((NC, ROWS, D), lambda i: (0, i, 0)),
            pl.BlockSpec((ROWS, D), lambda i: (i, 0)),
            pl.BlockSpec((ROWS, 1), lambda i: (i, 0)),
            pl.BlockSpec((1, D), lambda i: (0, 0)),
            pl.BlockSpec((D, dout), lambda i: (0, 0)),
        ],
        out_specs=pl.BlockSpec((ROWS, dout), lambda i: (i, 0)),
        out_shape=jax.ShapeDtypeStruct((out_rows, dout), jnp.float32),
    )(p, g, dinv, b, w)


def _tc_out(p, g2, dinv, b2):
    return pl.pallas_call(
        _tc_out_body,
        grid=(N // ROWS,),
        in_specs=[
            pl.BlockSpec((NC, ROWS, 1), lambda i: (0, i, 0)),
            pl.BlockSpec((ROWS, 1), lambda i: (i, 0)),
            pl.BlockSpec((ROWS, 1), lambda i: (i, 0)),
            pl.BlockSpec((1, 1), lambda i: (0, 0)),
        ],
        out_specs=pl.BlockSpec((ROWS, 1), lambda i: (i, 0)),
        out_shape=jax.ShapeDtypeStruct((N, 1), jnp.float32),
    )(p, g2, dinv, b2)


def kernel(x, edge_index, W0, b0, W1, b1, W2, b2):
    src = edge_index[0].astype(jnp.int32)
    dst = edge_index[1].astype(jnp.int32)
    pad = (jnp.arange(EP - E, dtype=jnp.int32) % 16) + N
    srcp = jnp.concatenate([src, pad])
    dstp = jnp.concatenate([dst, pad])
    zNP = jnp.zeros((NP,), jnp.float32)
    zNPD = jnp.zeros((NP, D), jnp.float32)

    degp = _deg_kernel(dstp, zNP)
    g0, dinv = _tc1(degp.reshape(NC, NP, 1), x, W0)
    p0 = _prop_kernel(g0, srcp, dstp, zNPD)
    g1 = _tc_layer(p0, g0, dinv, b0.reshape(1, D), W1)
    p1 = _prop_kernel(g1, srcp, dstp, zNPD)
    g2 = _tc_layer(p1, g1, dinv, b1.reshape(1, D), W2)
    g2p = jnp.concatenate([g2.reshape(N), jnp.zeros((16,), jnp.float32)])
    p2 = _prop1_kernel(g2p, srcp, dstp, zNP)
    out = _tc_out(p2.reshape(NC, NP, 1), g2, dinv, b2.reshape(1, 1))
    return out


# speedup vs baseline: 1.8947x; 1.0130x over previous
import functools

import jax
import jax.numpy as jnp
from jax import lax
from jax.experimental import pallas as pl
from jax.experimental.pallas import tpu as pltpu
from jax.experimental.pallas import tpu_sc as plsc

N = 10000
E = 320000
D = 128

NC = 2
NS = 16
NP = N + 112
RSPL = NP // NS
EPT = 10240
EP = EPT * NC * NS

_mesh = plsc.VectorSubcoreMesh(core_axis_name="c", subcore_axis_name="s")


def _ring3_edge_loop(nch, src_load, src_wait, dst_load, dst_wait,
                     g_start, g_wait, s_start, s_wait, after_prologue=None):
    def make_iter(k, b, static):
        def when(cond, fn):
            if static:
                if cond:
                    fn()
            else:
                pl.when(cond)(fn)

        when(k >= 1, lambda: s_wait((b + 2) % 3))
        when(k + 2 < nch, lambda: dst_load(k + 2, (b + 2) % 3))
        when(k + 2 < nch, lambda: src_wait((b + 2) % 3))
        when(k + 2 < nch, lambda: g_start(k + 2, (b + 2) % 3))
        g_wait(b)
        when(k + 3 < nch, lambda: src_load(k + 3, b))
        dst_wait(b)
        s_start(k, b)

    src_load(0, 0)
    dst_load(0, 0)
    src_load(1, 1)
    dst_load(1, 1)
    src_wait(0)
    src_wait(1)
    g_start(0, 0)
    g_start(1, 1)
    if nch > 2:
        src_load(2, 2)
    if after_prologue is not None:
        after_prologue()

    ngroups = nch // 3

    def body(j, carry):
        for t in range(3):
            make_iter(3 * j + t, t, static=False)
        return carry

    lax.fori_loop(0, ngroups, body, 0)
    for k in range(3 * ngroups, nch):
        make_iter(k, k % 3, static=True)
    s_wait((nch - 1) % 3)


def _edge_scatter_kernel(gv, src_hbm, dst_hbm, z_hbm, out_hbm,
                         src_v, dst_v, rows_v, sr, sd, sg, ss, sz,
                         acc_sh, pc, nch, c, s, split_io):
    if split_io:
        zinit = pltpu.async_copy(z_hbm.at[pl.ds(s * RSPL, RSPL)],
                                 acc_sh.at[pl.ds(s * RSPL, RSPL)], sz)
    else:
        zinit = None

        @pl.when(s == 0)
        def _():
            pltpu.async_copy(z_hbm, acc_sh, sz)

    def after_prologue():
        if split_io:
            zinit.wait()
        else:
            @pl.when(s == 0)
            def _():
                pltpu.make_async_copy(z_hbm, acc_sh, sz).wait()
        plsc.subcore_barrier()

    base0 = (c * NS + s) * EPT

    def src_load(k, b):
        pltpu.async_copy(src_hbm.at[pl.ds(base0 + k * pc, pc)], src_v[b], sr[b])

    def src_wait(b):
        pltpu.make_async_copy(src_hbm.at[pl.ds(base0, pc)], src_v[b], sr[b]).wait()

    def dst_load(k, b):
        pltpu.async_copy(dst_hbm.at[pl.ds(base0 + k * pc, pc)], dst_v[b], sd[b])

    def dst_wait(b):
        pltpu.make_async_copy(dst_hbm.at[pl.ds(base0, pc)], dst_v[b], sd[b]).wait()

    def g_start(k, b):
        pltpu.async_copy(gv.at[src_v[b]], rows_v[b], sg[b])

    def g_wait(b):
        pltpu.make_async_copy(gv.at[src_v[b]], rows_v[b], sg[b]).wait()

    def s_start(k, b):
        pltpu.async_copy(rows_v[b], acc_sh.at[dst_v[b]], ss[b], add=True)

    def s_wait(b):
        pltpu.make_async_copy(rows_v[b], acc_sh.at[dst_v[b]], ss[b]).wait()

    _ring3_edge_loop(nch, src_load, src_wait, dst_load, dst_wait,
                     g_start, g_wait, s_start, s_wait, after_prologue)

    plsc.subcore_barrier()
    if split_io:
        pltpu.sync_copy(acc_sh.at[pl.ds(s * RSPL, RSPL)],
                        out_hbm.at[c].at[pl.ds(s * RSPL, RSPL)])
    else:
        @pl.when(s == 0)
        def _():
            pltpu.sync_copy(acc_sh, out_hbm.at[c])


DEG_CHUNK = 512
DEG_NCHUNKS = EPT // DEG_CHUNK


@functools.partial(
    pl.kernel,
    mesh=_mesh,
    out_type=jax.ShapeDtypeStruct((NC, NP), jnp.float32),
    scratch_types=[
        pltpu.VMEM((DEG_CHUNK,), jnp.int32),
        pltpu.VMEM((DEG_CHUNK,), jnp.float32),
        pltpu.VMEM_SHARED((NP,), jnp.float32),
    ],
)
def _deg_kernel(dst_hbm, z_hbm, out_hbm, dst_v, ones_v, acc_sh):
    c = lax.axis_index("c")
    s = lax.axis_index("s")

    @pl.when(s == 0)
    def _():
        pltpu.sync_copy(z_hbm, acc_sh)

    def fill_ones(j, carry):
        ones_v[pl.ds(j * 16, 16)] = jnp.ones((16,), jnp.float32)
        return carry

    lax.fori_loop(0, DEG_CHUNK // 16, fill_ones, 0)
    plsc.subcore_barrier()

    base0 = (c * NS + s) * EPT

    def chunk(k, carry):
        base = base0 + k * DEG_CHUNK
        pltpu.sync_copy(dst_hbm.at[pl.ds(base, DEG_CHUNK)], dst_v)
        pltpu.sync_copy(ones_v, acc_sh.at[dst_v], add=True)
        return carry

    lax.fori_loop(0, DEG_NCHUNKS, chunk, 0)
    plsc.subcore_barrier()

    @pl.when(s == 0)
    def _():
        pltpu.sync_copy(acc_sh, out_hbm.at[c])


PC = 128
PNCH = EPT // PC


@functools.partial(
    pl.kernel,
    mesh=_mesh,
    out_type=jax.ShapeDtypeStruct((NC, NP, D), jnp.float32),
    scratch_types=(
        [pltpu.VMEM((PC,), jnp.int32)] * 3
        + [pltpu.VMEM((PC,), jnp.int32)] * 3
        + [pltpu.VMEM((PC, D), jnp.float32)] * 3
        + [pltpu.VMEM_SHARED((NP, D), jnp.float32)]
        + [pltpu.SemaphoreType.DMA] * 13
    ),
)
def _prop_kernel(g_hbm, src_hbm, dst_hbm, z_hbm, out_hbm,
                 s0, s1, s2, d0, d1, d2, r0, r1, r2, acc_sh, *sems):
    c = lax.axis_index("c")
    s = lax.axis_index("s")
    _edge_scatter_kernel(g_hbm, src_hbm, dst_hbm, z_hbm, out_hbm,
                         [s0, s1, s2], [d0, d1, d2], [r0, r1, r2],
                         list(sems[0:3]), list(sems[3:6]), list(sems[6:9]),
                         list(sems[9:12]), sems[12], acc_sh, PC, PNCH, c, s, split_io=True)


P1C = 1024
P1NCH = EPT // P1C


@functools.partial(
    pl.kernel,
    mesh=_mesh,
    out_type=jax.ShapeDtypeStruct((NC, NP), jnp.float32),
    scratch_types=(
        [pltpu.VMEM((P1C,), jnp.int32)] * 3
        + [pltpu.VMEM((P1C,), jnp.int32)] * 3
        + [pltpu.VMEM((P1C,), jnp.float32)] * 3
        + [pltpu.VMEM_SHARED((NP,), jnp.float32)]
        + [pltpu.SemaphoreType.DMA] * 13
    ),
)
def _prop1_kernel(g_hbm, src_hbm, dst_hbm, z_hbm, out_hbm,
                  s0, s1, s2, d0, d1, d2, r0, r1, r2, acc_sh, *sems):
    c = lax.axis_index("c")
    s = lax.axis_index("s")
    _edge_scatter_kernel(g_hbm, src_hbm, dst_hbm, z_hbm, out_hbm,
                         [s0, s1, s2], [d0, d1, d2], [r0, r1, r2],
                         list(sems[0:3]), list(sems[3:6]), list(sems[6:9]),
                         list(sems[9:12]), sems[12], acc_sh, P1C, P1NCH, c, s, split_io=False)


ROWS = 1000


def _tc1_body(degp_ref, x_ref, w_ref, g_ref, dinv_ref):
    d = degp_ref[0] + degp_ref[1] + 1.0
    dinv = lax.rsqrt(d)
    dinv_ref[...] = dinv
    h = jnp.dot(x_ref[...], w_ref[...], preferred_element_type=jnp.float32)
    g_ref[...] = h * dinv


def _tc_layer_body(p_ref, g_ref, dinv_ref, b_ref, w_ref, out_ref):
    s = p_ref[0] + p_ref[1] + g_ref[...]
    a = jnp.maximum(s * dinv_ref[...] + b_ref[...], 0.0)
    h = jnp.dot(a, w_ref[...], preferred_element_type=jnp.float32)
    out_ref[...] = h * dinv_ref[...]


def _tc_out_body(p_ref, g_ref, dinv_ref, b_ref, out_ref):
    out_ref[...] = (p_ref[0] + p_ref[1] + g_ref[...]) * dinv_ref[...] + b_ref[...]


def _tc1(degp, x, w0):
    return pl.pallas_call(
        _tc1_body,
        grid=(N // ROWS,),
        in_specs=[
            pl.BlockSpec((NC, ROWS, 1), lambda i: (0, i, 0)),
            pl.BlockSpec((ROWS, D), lambda i: (i, 0)),
            pl.BlockSpec((D, D), lambda i: (0, 0)),
        ],
        out_specs=[
            pl.BlockSpec((ROWS, D), lambda i: (i, 0)),
            pl.BlockSpec((ROWS, 1), lambda i: (i, 0)),
        ],
        out_shape=[
            jax.ShapeDtypeStruct((NP, D), jnp.float32),
            jax.ShapeDtypeStruct((N, 1), jnp.float32),
        ],
    )(degp, x, w0)


def _tc_layer(p, g, dinv, b, w):
    dout = w.shape[1]
    out_rows = NP if dout == D else N
    return pl.pallas_call(
        _tc_layer_body,
        grid=(N // ROWS,),
        in_specs=[
            pl.BlockSpec((NC, ROWS, D), lambda i: (0, i, 0)),
            pl.BlockSpec((ROWS, D), lambda i: (i, 0)),
            pl.BlockSpec((ROWS, 1), lambda i: (i, 0)),
            pl.BlockSpec((1, D), lambda i: (0, 0)),
            pl.BlockSpec((D, dout), lambda i: (0, 0)),
        ],
        out_specs=pl.BlockSpec((ROWS, dout), lambda i: (i, 0)),
        out_shape=jax.ShapeDtypeStruct((out_rows, dout), jnp.float32),
    )(p, g, dinv, b, w)


def _tc_out(p, g2, dinv, b2):
    return pl.pallas_call(
        _tc_out_body,
        grid=(N // ROWS,),
        in_specs=[
            pl.BlockSpec((NC, ROWS, 1), lambda i: (0, i, 0)),
            pl.BlockSpec((ROWS, 1), lambda i: (i, 0)),
            pl.BlockSpec((ROWS, 1), lambda i: (i, 0)),
            pl.BlockSpec((1, 1), lambda i: (0, 0)),
        ],
        out_specs=pl.BlockSpec((ROWS, 1), lambda i: (i, 0)),
        out_shape=jax.ShapeDtypeStruct((N, 1), jnp.float32),
    )(p, g2, dinv, b2)


def kernel(x, edge_index, W0, b0, W1, b1, W2, b2):
    src = edge_index[0].astype(jnp.int32)
    dst = edge_index[1].astype(jnp.int32)
    pad = (jnp.arange(EP - E, dtype=jnp.int32) % 16) + N
    srcp = jnp.concatenate([src, pad])
    dstp = jnp.concatenate([dst, pad])
    zNP = jnp.zeros((NP,), jnp.float32)
    zNPD = jnp.zeros((NP, D), jnp.float32)

    degp = _deg_kernel(dstp, zNP)
    g0, dinv = _tc1(degp.reshape(NC, NP, 1), x, W0)
    p0 = _prop_kernel(g0, srcp, dstp, zNPD)
    g1 = _tc_layer(p0, g0, dinv, b0.reshape(1, D), W1)
    p1 = _prop_kernel(g1, srcp, dstp, zNPD)
    g2 = _tc_layer(p1, g1, dinv, b1.reshape(1, D), W2)
    g2p = jnp.concatenate([g2.reshape(N), jnp.zeros((16,), jnp.float32)])
    p2 = _prop1_kernel(g2p, srcp, dstp, zNP)
    out = _tc_out(p2.reshape(NC, NP, 1), g2, dinv, b2.reshape(1, 1))
    return out


# speedup vs baseline: 1.8996x; 1.0026x over previous
import functools

import jax
import jax.numpy as jnp
from jax import lax
from jax.experimental import pallas as pl
from jax.experimental.pallas import tpu as pltpu
from jax.experimental.pallas import tpu_sc as plsc

N = 10000
E = 320000
D = 128

NC = 2
NS = 16
NP = N + 112
RSPL = NP // NS
EPT = 10240
EP = EPT * NC * NS

_mesh = plsc.VectorSubcoreMesh(core_axis_name="c", subcore_axis_name="s")


def _ring3_edge_loop(nch, src_load, src_wait, dst_load, dst_wait,
                     g_start, g_wait, s_start, s_wait, after_prologue=None):
    def make_iter(k, b, static):
        def when(cond, fn):
            if static:
                if cond:
                    fn()
            else:
                pl.when(cond)(fn)

        when(k >= 1, lambda: s_wait((b + 2) % 3))
        when(k + 2 < nch, lambda: dst_load(k + 2, (b + 2) % 3))
        when(k + 2 < nch, lambda: src_wait((b + 2) % 3))
        when(k + 2 < nch, lambda: g_start(k + 2, (b + 2) % 3))
        g_wait(b)
        when(k + 3 < nch, lambda: src_load(k + 3, b))
        dst_wait(b)
        s_start(k, b)

    src_load(0, 0)
    dst_load(0, 0)
    src_load(1, 1)
    dst_load(1, 1)
    src_wait(0)
    src_wait(1)
    g_start(0, 0)
    g_start(1, 1)
    if nch > 2:
        src_load(2, 2)
    if after_prologue is not None:
        after_prologue()

    ngroups = nch // 3

    def body(j, carry):
        for t in range(3):
            make_iter(3 * j + t, t, static=False)
        return carry

    lax.fori_loop(0, ngroups, body, 0)
    for k in range(3 * ngroups, nch):
        make_iter(k, k % 3, static=True)
    s_wait((nch - 1) % 3)


def _edge_scatter_kernel(gv, src_hbm, dst_hbm, z_hbm, out_hbm,
                         src_v, dst_v, rows_v, sr, sd, sg, ss, sz,
                         acc_sh, pc, nch, c, s, split_io):
    if split_io:
        zinit = pltpu.async_copy(z_hbm.at[pl.ds(s * RSPL, RSPL)],
                                 acc_sh.at[pl.ds(s * RSPL, RSPL)], sz)
    else:
        zinit = None

        @pl.when(s == 0)
        def _():
            pltpu.async_copy(z_hbm, acc_sh, sz)

    def after_prologue():
        if split_io:
            zinit.wait()
        else:
            @pl.when(s == 0)
            def _():
                pltpu.make_async_copy(z_hbm, acc_sh, sz).wait()
        plsc.subcore_barrier()

    base0 = (c * NS + s) * EPT

    def src_load(k, b):
        pltpu.async_copy(src_hbm.at[pl.ds(base0 + k * pc, pc)], src_v[b], sr[b])

    def src_wait(b):
        pltpu.make_async_copy(src_hbm.at[pl.ds(base0, pc)], src_v[b], sr[b]).wait()

    def dst_load(k, b):
        pltpu.async_copy(dst_hbm.at[pl.ds(base0 + k * pc, pc)], dst_v[b], sd[b])

    def dst_wait(b):
        pltpu.make_async_copy(dst_hbm.at[pl.ds(base0, pc)], dst_v[b], sd[b]).wait()

    def g_start(k, b):
        pltpu.async_copy(gv.at[src_v[b]], rows_v[b], sg[b])

    def g_wait(b):
        pltpu.make_async_copy(gv.at[src_v[b]], rows_v[b], sg[b]).wait()

    def s_start(k, b):
        pltpu.async_copy(rows_v[b], acc_sh.at[dst_v[b]], ss[b], add=True)

    def s_wait(b):
        pltpu.make_async_copy(rows_v[b], acc_sh.at[dst_v[b]], ss[b]).wait()

    _ring3_edge_loop(nch, src_load, src_wait, dst_load, dst_wait,
                     g_start, g_wait, s_start, s_wait, after_prologue)

    plsc.subcore_barrier()
    if split_io:
        pltpu.sync_copy(acc_sh.at[pl.ds(s * RSPL, RSPL)],
                        out_hbm.at[c].at[pl.ds(s * RSPL, RSPL)])
    else:
        @pl.when(s == 0)
        def _():
            pltpu.sync_copy(acc_sh, out_hbm.at[c])


DEG_CHUNK = 512
DEG_NCHUNKS = EPT // DEG_CHUNK


@functools.partial(
    pl.kernel,
    mesh=_mesh,
    out_type=jax.ShapeDtypeStruct((NC, NP), jnp.float32),
    scratch_types=[
        pltpu.VMEM((DEG_CHUNK,), jnp.int32),
        pltpu.VMEM((DEG_CHUNK,), jnp.float32),
        pltpu.VMEM_SHARED((NP,), jnp.float32),
    ],
)
def _deg_kernel(dst_hbm, z_hbm, out_hbm, dst_v, ones_v, acc_sh):
    c = lax.axis_index("c")
    s = lax.axis_index("s")

    @pl.when(s == 0)
    def _():
        pltpu.sync_copy(z_hbm, acc_sh)

    def fill_ones(j, carry):
        ones_v[pl.ds(j * 16, 16)] = jnp.ones((16,), jnp.float32)
        return carry

    lax.fori_loop(0, DEG_CHUNK // 16, fill_ones, 0)
    plsc.subcore_barrier()

    base0 = (c * NS + s) * EPT

    def chunk(k, carry):
        base = base0 + k * DEG_CHUNK
        pltpu.sync_copy(dst_hbm.at[pl.ds(base, DEG_CHUNK)], dst_v)
        pltpu.sync_copy(ones_v, acc_sh.at[dst_v], add=True)
        return carry

    lax.fori_loop(0, DEG_NCHUNKS, chunk, 0)
    plsc.subcore_barrier()

    @pl.when(s == 0)
    def _():
        pltpu.sync_copy(acc_sh, out_hbm.at[c])


PC = 128
PNCH = EPT // PC


@functools.partial(
    pl.kernel,
    mesh=_mesh,
    out_type=jax.ShapeDtypeStruct((NC, NP, D), jnp.float32),
    scratch_types=(
        [pltpu.VMEM((PC,), jnp.int32)] * 3
        + [pltpu.VMEM((PC,), jnp.int32)] * 3
        + [pltpu.VMEM((PC, D), jnp.float32)] * 3
        + [pltpu.VMEM_SHARED((NP, D), jnp.float32)]
        + [pltpu.SemaphoreType.DMA] * 13
    ),
)
def _prop_kernel(g_hbm, src_hbm, dst_hbm, z_hbm, out_hbm,
                 s0, s1, s2, d0, d1, d2, r0, r1, r2, acc_sh, *sems):
    c = lax.axis_index("c")
    s = lax.axis_index("s")
    _edge_scatter_kernel(g_hbm, src_hbm, dst_hbm, z_hbm, out_hbm,
                         [s0, s1, s2], [d0, d1, d2], [r0, r1, r2],
                         list(sems[0:3]), list(sems[3:6]), list(sems[6:9]),
                         list(sems[9:12]), sems[12], acc_sh, PC, PNCH, c, s, split_io=True)


P1C = 2048
P1NCH = EPT // P1C


@functools.partial(
    pl.kernel,
    mesh=_mesh,
    out_type=jax.ShapeDtypeStruct((NC, NP), jnp.float32),
    scratch_types=(
        [pltpu.VMEM((P1C,), jnp.int32)] * 3
        + [pltpu.VMEM((P1C,), jnp.int32)] * 3
        + [pltpu.VMEM((P1C,), jnp.float32)] * 3
        + [pltpu.VMEM_SHARED((NP,), jnp.float32)]
        + [pltpu.SemaphoreType.DMA] * 13
    ),
)
def _prop1_kernel(g_hbm, src_hbm, dst_hbm, z_hbm, out_hbm,
                  s0, s1, s2, d0, d1, d2, r0, r1, r2, acc_sh, *sems):
    c = lax.axis_index("c")
    s = lax.axis_index("s")
    _edge_scatter_kernel(g_hbm, src_hbm, dst_hbm, z_hbm, out_hbm,
                         [s0, s1, s2], [d0, d1, d2], [r0, r1, r2],
                         list(sems[0:3]), list(sems[3:6]), list(sems[6:9]),
                         list(sems[9:12]), sems[12], acc_sh, P1C, P1NCH, c, s, split_io=False)


ROWS = 1000


def _tc1_body(degp_ref, x_ref, w_ref, g_ref, dinv_ref):
    d = degp_ref[0] + degp_ref[1] + 1.0
    dinv = lax.rsqrt(d)
    dinv_ref[...] = dinv
    h = jnp.dot(x_ref[...], w_ref[...], preferred_element_type=jnp.float32)
    g_ref[...] = h * dinv


def _tc_layer_body(p_ref, g_ref, dinv_ref, b_ref, w_ref, out_ref):
    s = p_ref[0] + p_ref[1] + g_ref[...]
    a = jnp.maximum(s * dinv_ref[...] + b_ref[...], 0.0)
    h = jnp.dot(a, w_ref[...], preferred_element_type=jnp.float32)
    out_ref[...] = h * dinv_ref[...]


def _tc_out_body(p_ref, g_ref, dinv_ref, b_ref, out_ref):
    out_ref[...] = (p_ref[0] + p_ref[1] + g_ref[...]) * dinv_ref[...] + b_ref[...]


def _tc1(degp, x, w0):
    return pl.pallas_call(
        _tc1_body,
        grid=(N // ROWS,),
        in_specs=[
            pl.BlockSpec((NC, ROWS, 1), lambda i: (0, i, 0)),
            pl.BlockSpec((ROWS, D), lambda i: (i, 0)),
            pl.BlockSpec((D, D), lambda i: (0, 0)),
        ],
        out_specs=[
            pl.BlockSpec((ROWS, D), lambda i: (i, 0)),
            pl.BlockSpec((ROWS, 1), lambda i: (i, 0)),
        ],
        out_shape=[
            jax.ShapeDtypeStruct((NP, D), jnp.float32),
            jax.ShapeDtypeStruct((N, 1), jnp.float32),
        ],
    )(degp, x, w0)


def _tc_layer(p, g, dinv, b, w):
    dout = w.shape[1]
    out_rows = NP if dout == D else N
    return pl.pallas_call(
        _tc_layer_body,
        grid=(N // ROWS,),
        in_specs=[
            pl.BlockSpec((NC, ROWS, D), lambda i: (0, i, 0)),
            pl.BlockSpec((ROWS, D), lambda i: (i, 0)),
            pl.BlockSpec((ROWS, 1), lambda i: (i, 0)),
            pl.BlockSpec((1, D), lambda i: (0, 0)),
            pl.BlockSpec((D, dout), lambda i: (0, 0)),
        ],
        out_specs=pl.BlockSpec((ROWS, dout), lambda i: (i, 0)),
        out_shape=jax.ShapeDtypeStruct((out_rows, dout), jnp.float32),
    )(p, g, dinv, b, w)


def _tc_out(p, g2, dinv, b2):
    return pl.pallas_call(
        _tc_out_body,
        grid=(N // ROWS,),
        in_specs=[
            pl.BlockSpec((NC, ROWS, 1), lambda i: (0, i, 0)),
            pl.BlockSpec((ROWS, 1), lambda i: (i, 0)),
            pl.BlockSpec((ROWS, 1), lambda i: (i, 0)),
            pl.BlockSpec((1, 1), lambda i: (0, 0)),
        ],
        out_specs=pl.BlockSpec((ROWS, 1), lambda i: (i, 0)),
        out_shape=jax.ShapeDtypeStruct((N, 1), jnp.float32),
    )(p, g2, dinv, b2)


def kernel(x, edge_index, W0, b0, W1, b1, W2, b2):
    src = edge_index[0].astype(jnp.int32)
    dst = edge_index[1].astype(jnp.int32)
    pad = (jnp.arange(EP - E, dtype=jnp.int32) % 16) + N
    srcp = jnp.concatenate([src, pad])
    dstp = jnp.concatenate([dst, pad])
    zNP = jnp.zeros((NP,), jnp.float32)
    zNPD = jnp.zeros((NP, D), jnp.float32)

    degp = _deg_kernel(dstp, zNP)
    g0, dinv = _tc1(degp.reshape(NC, NP, 1), x, W0)
    p0 = _prop_kernel(g0, srcp, dstp, zNPD)
    g1 = _tc_layer(p0, g0, dinv, b0.reshape(1, D), W1)
    p1 = _prop_kernel(g1, srcp, dstp, zNPD)
    g2 = _tc_layer(p1, g1, dinv, b1.reshape(1, D), W2)
    g2p = jnp.concatenate([g2.reshape(N), jnp.zeros((16,), jnp.float32)])
    p2 = _prop1_kernel(g2p, srcp, dstp, zNP)
    out = _tc_out(p2.reshape(NC, NP, 1), g2, dinv, b2.reshape(1, 1))
    return out
